# baseline (speedup 1.0000x reference)
import functools

import jax
import jax.numpy as jnp
from jax import lax
from jax.experimental import pallas as pl
from jax.experimental.pallas import tpu as pltpu
from jax.experimental.pallas import tpu_sc as plsc

_TOPK = 8

_SORT8 = [(0, 1), (2, 3), (4, 5), (6, 7),
          (0, 2), (1, 3), (4, 6), (5, 7),
          (1, 2), (5, 6),
          (0, 4), (1, 5), (2, 6), (3, 7),
          (1, 4), (3, 6),
          (2, 4), (3, 5),
          (3, 4)]
_BMERGE8 = [(0, 4), (1, 5), (2, 6), (3, 7),
            (0, 2), (1, 3), (4, 6), (5, 7),
            (0, 1), (2, 3), (4, 5), (6, 7)]


def _cas(p, q):
    pv, pi = p
    qv, qi = q
    c = pv >= qv
    hi = (jnp.where(c, pv, qv), jnp.where(c, pi, qi))
    lo = (jnp.where(c, qv, pv), jnp.where(c, qi, pi))
    return hi, lo


def _sort8(pairs):
    pairs = list(pairs)
    for a, b in _SORT8:
        pairs[a], pairs[b] = _cas(pairs[a], pairs[b])
    return pairs


def _merge_top8(A, B):
    c = []
    for i in range(8):
        pv, pi = A[i]
        qv, qi = B[7 - i]
        m = pv >= qv
        c.append((jnp.where(m, pv, qv), jnp.where(m, pi, qi)))
    for a, b in _BMERGE8:
        c[a], c[b] = _cas(c[a], c[b])
    return c


def _matmul_body(x_ref, w_ref, b_ref, lt_ref):
    lt = jax.lax.dot_general(
        w_ref[...], x_ref[...], (((1,), (1,)), ((), ())),
        preferred_element_type=jnp.float32,
    )
    lt_ref[...] = lt + b_ref[...]


def _logits_t(x, W, b, tile, row0, nrows):
    d = x.shape[1]
    e = W.shape[0]
    base = row0 // tile
    return pl.pallas_call(
        _matmul_body,
        grid=(nrows // tile,),
        in_specs=[
            pl.BlockSpec((tile, d), lambda i: (base + i, 0)),
            pl.BlockSpec((e, d), lambda i: (0, 0)),
            pl.BlockSpec((e, 1), lambda i: (0, 0)),
        ],
        out_specs=pl.BlockSpec((e, tile), lambda i: (0, i)),
        out_shape=jax.ShapeDtypeStruct((e, nrows), jnp.float32),
    )(x, W, b.reshape(e, 1))


def _make_sc_topk(n, e):
    info = plsc.get_sparse_core_info()
    nc, ns, nl = info.num_cores, info.num_subcores, info.num_lanes
    nw = nc * ns
    assert n % (nw * nl) == 0 and e == 64
    tok_w = n // nw
    ngroups = tok_w // nl
    mesh = plsc.VectorSubcoreMesh(core_axis_name="c", subcore_axis_name="s")

    @functools.partial(
        pl.kernel, mesh=mesh,
        out_type=[
            jax.ShapeDtypeStruct((_TOPK, n), jnp.float32),
            jax.ShapeDtypeStruct((_TOPK, n), jnp.int32),
        ],
        scratch_types=[
            pltpu.VMEM((e, tok_w), jnp.float32),
            pltpu.VMEM((_TOPK, tok_w), jnp.float32),
            pltpu.VMEM((_TOPK, tok_w), jnp.int32),
        ],
    )
    def sc_topk(lt_hbm, gt_hbm, it_hbm, lt_v, g_v, i_v):
        wid = lax.axis_index("s") * nc + lax.axis_index("c")
        base = wid * tok_w
        pltpu.sync_copy(lt_hbm.at[:, pl.ds(base, tok_w)], lt_v)

        def group_body(g, carry):
            off = g * nl

            def sorted_group(j):
                pairs = [
                    (lt_v[8 * j + t, pl.ds(off, nl)],
                     jnp.full((nl,), 8 * j + t, jnp.int32))
                    for t in range(8)
                ]
                return _sort8(pairs)

            top = sorted_group(0)
            for j in range(1, 8):
                top = _merge_top8(top, sorted_group(j))

            m = top[0][0]
            exps = [jnp.exp(tv - m) for tv, _ in top]
            denom = exps[0]
            for s in exps[1:]:
                denom = denom + s
            inv = 1.0 / denom
            for k in range(_TOPK):
                g_v[k, pl.ds(off, nl)] = exps[k] * inv
                i_v[k, pl.ds(off, nl)] = top[k][1]
            return carry

        lax.fori_loop(0, ngroups, group_body, 0)
        pltpu.sync_copy(g_v, gt_hbm.at[:, pl.ds(base, tok_w)])
        pltpu.sync_copy(i_v, it_hbm.at[:, pl.ds(base, tok_w)])

    return sc_topk


def kernel(x, W, b):
    n, d = x.shape
    e = W.shape[0]
    tile = 1024 if n % 1024 == 0 else n
    nchunks = 4
    if n % nchunks or (n // nchunks) % (32 * 16) or (n // nchunks) % tile:
        nchunks = 1
    crows = n // nchunks
    sc_topk = _make_sc_topk(crows, e)
    gts, its = [], []
    for ci in range(nchunks):
        lt = _logits_t(x, W, b, tile, ci * crows, crows)
        gt, it = sc_topk(lt)
        gts.append(gt)
        its.append(it)
    gt = jnp.concatenate(gts, axis=1) if nchunks > 1 else gts[0]
    it = jnp.concatenate(its, axis=1) if nchunks > 1 else its[0]
    return gt.T, it.T.astype(jnp.int64)


# speedup vs baseline: 1.0795x; 1.0795x over previous
import functools

import jax
import jax.numpy as jnp
from jax import lax
from jax.experimental import pallas as pl
from jax.experimental.pallas import tpu as pltpu
from jax.experimental.pallas import tpu_sc as plsc

_TOPK = 8

_SORT8 = [(0, 1), (2, 3), (4, 5), (6, 7),
          (0, 2), (1, 3), (4, 6), (5, 7),
          (1, 2), (5, 6),
          (0, 4), (1, 5), (2, 6), (3, 7),
          (1, 4), (3, 6),
          (2, 4), (3, 5),
          (3, 4)]
_BMERGE8 = [(0, 4), (1, 5), (2, 6), (3, 7),
            (0, 2), (1, 3), (4, 6), (5, 7),
            (0, 1), (2, 3), (4, 5), (6, 7)]


def _sort8(v):
    v = list(v)
    for a, b in _SORT8:
        v[a], v[b] = jnp.maximum(v[a], v[b]), jnp.minimum(v[a], v[b])
    return v


def _merge_top8(A, B):
    c = [jnp.maximum(A[i], B[7 - i]) for i in range(8)]
    for a, b in _BMERGE8:
        c[a], c[b] = jnp.maximum(c[a], c[b]), jnp.minimum(c[a], c[b])
    return c


def _matmul_pack_body(x_ref, w_ref, b_ref, key_ref):
    lt = jax.lax.dot_general(
        w_ref[...], x_ref[...], (((1,), (1,)), ((), ())),
        preferred_element_type=jnp.float32,
    )
    lt = lt + b_ref[...]
    s = jax.lax.bitcast_convert_type(lt, jnp.int32)
    u = jax.lax.bitcast_convert_type(
        s ^ ((s >> 31) | jnp.int32(-(2 ** 31))), jnp.uint32
    )
    eid = jax.lax.broadcasted_iota(jnp.uint32, lt.shape, 0)
    key_ref[...] = (u & jnp.uint32(0xFFFFFFC0)) | (jnp.uint32(63) - eid)


def _packed_keys_t(x, W, b, tile):
    n, d = x.shape
    e = W.shape[0]
    return pl.pallas_call(
        _matmul_pack_body,
        grid=(n // tile,),
        in_specs=[
            pl.BlockSpec((tile, d), lambda i: (i, 0)),
            pl.BlockSpec((e, d), lambda i: (0, 0)),
            pl.BlockSpec((e, 1), lambda i: (0, 0)),
        ],
        out_specs=pl.BlockSpec((e, tile), lambda i: (0, i)),
        out_shape=jax.ShapeDtypeStruct((e, n), jnp.uint32),
    )(x, W, b.reshape(e, 1))


def _make_sc_topk(n, e):
    info = plsc.get_sparse_core_info()
    nc, ns, nl = info.num_cores, info.num_subcores, info.num_lanes
    nw = nc * ns
    assert n % (nw * nl) == 0 and e == 64
    tok_w = n // nw
    ngroups = tok_w // nl
    mesh = plsc.VectorSubcoreMesh(core_axis_name="c", subcore_axis_name="s")

    @functools.partial(
        pl.kernel, mesh=mesh,
        out_type=[
            jax.ShapeDtypeStruct((_TOPK, n), jnp.float32),
            jax.ShapeDtypeStruct((_TOPK, n), jnp.int32),
        ],
        scratch_types=[
            pltpu.VMEM((e, tok_w), jnp.uint32),
            pltpu.VMEM((_TOPK, tok_w), jnp.float32),
            pltpu.VMEM((_TOPK, tok_w), jnp.int32),
        ],
    )
    def sc_topk(key_hbm, gt_hbm, it_hbm, key_v, g_v, i_v):
        wid = lax.axis_index("s") * nc + lax.axis_index("c")
        base = wid * tok_w
        pltpu.sync_copy(key_hbm.at[:, pl.ds(base, tok_w)], key_v)

        def group_body(g, carry):
            off = g * nl

            top = _sort8([key_v[t, pl.ds(off, nl)] for t in range(8)])
            for j in range(1, 8):
                top = _merge_top8(
                    top, _sort8([key_v[8 * j + t, pl.ds(off, nl)]
                                 for t in range(8)])
                )

            vals, idxs = [], []
            for k in top:
                idxs.append((jnp.uint32(63) - (k & jnp.uint32(63)))
                            .astype(jnp.int32))
                vu = (k & jnp.uint32(0xFFFFFFC0)) | jnp.uint32(32)
                pos = vu >= jnp.uint32(0x80000000)
                sb = jnp.where(pos, vu ^ jnp.uint32(0x80000000), ~vu)
                vals.append(jax.lax.bitcast_convert_type(sb, jnp.float32))

            m = vals[0]
            exps = [jnp.exp(v - m) for v in vals]
            denom = exps[0]
            for s in exps[1:]:
                denom = denom + s
            inv = 1.0 / denom
            for k in range(_TOPK):
                g_v[k, pl.ds(off, nl)] = exps[k] * inv
                i_v[k, pl.ds(off, nl)] = idxs[k]
            return carry

        lax.fori_loop(0, ngroups, group_body, 0)
        pltpu.sync_copy(g_v, gt_hbm.at[:, pl.ds(base, tok_w)])
        pltpu.sync_copy(i_v, it_hbm.at[:, pl.ds(base, tok_w)])

    return sc_topk


def kernel(x, W, b):
    n, d = x.shape
    e = W.shape[0]
    tile = 1024 if n % 1024 == 0 else n
    keys = _packed_keys_t(x, W, b, tile)
    gt, it = _make_sc_topk(n, e)(keys)
    return gt.T, it.T.astype(jnp.int64)
